# baseline (speedup 1.0000x reference)
import jax
import jax.numpy as jnp
from jax.experimental import pallas as pl
from jax.experimental.pallas import tpu as pltpu

N_NODES = 50000
D_FEAT = 512
NUM_SEGMENTS = 256
BN = 2000
NB = N_NODES // BN
NEG = -1e30


def _pool_kernel(feat_ref, seg_ref, wg_ref, wf_ref, bf_ref, out_ref,
                 m_run, d_run, acc):
    k = pl.program_id(0)

    @pl.when(k == 0)
    def _init():
        m_run[0, 0] = NEG
        d_run[...] = jnp.zeros((1, NUM_SEGMENTS), jnp.float32)
        acc[...] = jnp.zeros((D_FEAT, NUM_SEGMENTS), jnp.float32)

    feat = feat_ref[...]
    seg = seg_ref[0]
    g = jax.lax.dot_general(feat, wg_ref[...],
                            (((1,), (0,)), ((), ())),
                            preferred_element_type=jnp.float32)
    m_old = m_run[0, 0]
    m_new = jnp.maximum(m_old, jnp.max(g))

    @pl.when(m_new > m_old)
    def _rescale():
        s_old = jnp.exp(m_old - m_new)
        d_run[...] = d_run[...] * s_old
        acc[...] = acc[...] * s_old
        m_run[0, 0] = m_new

    e = jnp.exp(g - m_new)
    cols = jax.lax.broadcasted_iota(jnp.int32, (BN, NUM_SEGMENTS), 1)
    w = jnp.where(seg == cols, e, 0.0)
    d_run[...] += jnp.sum(w, axis=0, keepdims=True)
    acc[...] += jax.lax.dot_general(feat, w, (((0,), (0,)), ((), ())),
                                    preferred_element_type=jnp.float32)

    @pl.when(k == NB - 1)
    def _finish():
        d = d_run[...]
        inv = jnp.where(d > 0.0, 1.0 / d, 0.0)
        pooledT = acc[...] * inv
        out = jax.lax.dot_general(pooledT, wf_ref[...],
                                  (((0,), (0,)), ((), ())),
                                  preferred_element_type=jnp.float32)
        r = jax.lax.broadcasted_iota(jnp.int32, (NUM_SEGMENTS, NUM_SEGMENTS), 0)
        c = jax.lax.broadcasted_iota(jnp.int32, (NUM_SEGMENTS, NUM_SEGMENTS), 1)
        d_col = jnp.sum(jnp.where(r == c, jnp.broadcast_to(d, (NUM_SEGMENTS, NUM_SEGMENTS)), 0.0),
                        axis=1, keepdims=True)
        out_ref[...] = out + jnp.where(d_col > 0.0, bf_ref[...], 0.0)


@jax.jit
def kernel(feat, Wg, bg, Wf, bf, segment_ids):
    del bg
    seg3 = segment_ids.astype(jnp.int32).reshape(NB, BN, 1)
    bf2 = bf.reshape(1, D_FEAT)
    out = pl.pallas_call(
        _pool_kernel,
        grid=(NB,),
        in_specs=[
            pl.BlockSpec((BN, D_FEAT), lambda k: (k, 0)),
            pl.BlockSpec((1, BN, 1), lambda k: (k, 0, 0)),
            pl.BlockSpec((D_FEAT, 1), lambda k: (0, 0)),
            pl.BlockSpec((D_FEAT, D_FEAT), lambda k: (0, 0)),
            pl.BlockSpec((1, D_FEAT), lambda k: (0, 0)),
        ],
        out_specs=pl.BlockSpec((NUM_SEGMENTS, D_FEAT), lambda k: (0, 0)),
        out_shape=jax.ShapeDtypeStruct((NUM_SEGMENTS, D_FEAT), jnp.float32),
        scratch_shapes=[
            pltpu.SMEM((1, 1), jnp.float32),
            pltpu.VMEM((1, NUM_SEGMENTS), jnp.float32),
            pltpu.VMEM((D_FEAT, NUM_SEGMENTS), jnp.float32),
        ],
    )(feat, seg3, Wg, Wf, bf2)
    return out


# speedup vs baseline: 1.0691x; 1.0691x over previous
import jax
import jax.numpy as jnp
from jax.experimental import pallas as pl
from jax.experimental.pallas import tpu as pltpu

N_NODES = 50000
D_FEAT = 512
NUM_SEGMENTS = 256
BN = 2000
NB = N_NODES // BN
NEG = -1e30


def _pool_kernel(feat_ref, seg_ref, wg_ref, wf_ref, bf_ref, out_ref,
                 m_run, d_run, acc):
    k = pl.program_id(0)

    @pl.when(k == 0)
    def _init():
        m_run[0, 0] = NEG
        d_run[...] = jnp.zeros((1, NUM_SEGMENTS), jnp.float32)
        acc[...] = jnp.zeros((D_FEAT, NUM_SEGMENTS), jnp.float32)

    feat = feat_ref[...].astype(jnp.bfloat16)
    seg = seg_ref[0]
    g = jax.lax.dot_general(feat, wg_ref[...],
                            (((1,), (0,)), ((), ())),
                            preferred_element_type=jnp.float32)
    m_old = m_run[0, 0]
    m_new = jnp.maximum(m_old, jnp.max(g))

    @pl.when(m_new > m_old)
    def _rescale():
        s_old = jnp.exp(m_old - m_new)
        d_run[...] = d_run[...] * s_old
        acc[...] = acc[...] * s_old
        m_run[0, 0] = m_new

    e = jnp.exp(g - m_new)
    cols = jax.lax.broadcasted_iota(jnp.int32, (BN, NUM_SEGMENTS), 1)
    w32 = jnp.where(seg == cols, e, 0.0)
    w = w32.astype(jnp.bfloat16)
    d_run[...] += jnp.sum(w32, axis=0, keepdims=True)
    acc[...] += jax.lax.dot_general(feat, w, (((0,), (0,)), ((), ())),
                                    preferred_element_type=jnp.float32)

    @pl.when(k == NB - 1)
    def _finish():
        d = d_run[...]
        inv = jnp.where(d > 0.0, 1.0 / d, 0.0)
        pooledT = acc[...] * inv
        out = jax.lax.dot_general(pooledT, wf_ref[...],
                                  (((0,), (0,)), ((), ())),
                                  preferred_element_type=jnp.float32)
        r = jax.lax.broadcasted_iota(jnp.int32, (NUM_SEGMENTS, NUM_SEGMENTS), 0)
        c = jax.lax.broadcasted_iota(jnp.int32, (NUM_SEGMENTS, NUM_SEGMENTS), 1)
        d_col = jnp.sum(jnp.where(r == c, jnp.broadcast_to(d, (NUM_SEGMENTS, NUM_SEGMENTS)), 0.0),
                        axis=1, keepdims=True)
        out_ref[...] = out + jnp.where(d_col > 0.0, bf_ref[...], 0.0)


@jax.jit
def kernel(feat, Wg, bg, Wf, bf, segment_ids):
    del bg
    seg3 = segment_ids.astype(jnp.int32).reshape(NB, BN, 1)
    bf2 = bf.reshape(1, D_FEAT)
    Wg = Wg.astype(jnp.bfloat16)
    out = pl.pallas_call(
        _pool_kernel,
        grid=(NB,),
        in_specs=[
            pl.BlockSpec((BN, D_FEAT), lambda k: (k, 0)),
            pl.BlockSpec((1, BN, 1), lambda k: (k, 0, 0)),
            pl.BlockSpec((D_FEAT, 1), lambda k: (0, 0)),
            pl.BlockSpec((D_FEAT, D_FEAT), lambda k: (0, 0)),
            pl.BlockSpec((1, D_FEAT), lambda k: (0, 0)),
        ],
        out_specs=pl.BlockSpec((NUM_SEGMENTS, D_FEAT), lambda k: (0, 0)),
        out_shape=jax.ShapeDtypeStruct((NUM_SEGMENTS, D_FEAT), jnp.float32),
        scratch_shapes=[
            pltpu.SMEM((1, 1), jnp.float32),
            pltpu.VMEM((1, NUM_SEGMENTS), jnp.float32),
            pltpu.VMEM((D_FEAT, NUM_SEGMENTS), jnp.float32),
        ],
    )(feat, seg3, Wg, Wf, bf2)
    return out


# speedup vs baseline: 1.5278x; 1.4292x over previous
import jax
import jax.numpy as jnp
from jax.experimental import pallas as pl
from jax.experimental.pallas import tpu as pltpu

N_NODES = 50000
D_FEAT = 512
NUM_SEGMENTS = 256
BN = 2000
NB = N_NODES // BN
NEG = -1e30


def _pool_kernel(feat_ref, seg_ref, wg_ref, wf_ref, bf_ref, out_ref,
                 m_run, d_run, acc):
    k = pl.program_id(0)

    @pl.when(k == 0)
    def _init():
        m_run[0, 0] = NEG
        d_run[...] = jnp.zeros((NUM_SEGMENTS, 1), jnp.float32)
        acc[...] = jnp.zeros((NUM_SEGMENTS, D_FEAT), jnp.float32)

    feat = feat_ref[...].astype(jnp.bfloat16)
    seg = seg_ref[0]
    g_col = jax.lax.dot_general(feat, wg_ref[...],
                                (((1,), (0,)), ((), ())),
                                preferred_element_type=jnp.float32)
    g = jax.lax.transpose(g_col, (1, 0))
    m_old = m_run[0, 0]
    m_new = jnp.maximum(m_old, jnp.max(g))

    @pl.when(m_new > m_old)
    def _rescale():
        s_old = jnp.exp(m_old - m_new)
        d_run[...] = d_run[...] * s_old
        acc[...] = acc[...] * s_old
        m_run[0, 0] = m_new

    e = jnp.exp(g - m_new)
    rows = jax.lax.broadcasted_iota(jnp.int32, (NUM_SEGMENTS, BN), 0)
    w32 = jnp.where(seg == rows, e, 0.0)
    w = w32.astype(jnp.bfloat16)
    d_run[...] += jnp.sum(w32, axis=1, keepdims=True)
    acc[...] += jax.lax.dot_general(w, feat, (((1,), (0,)), ((), ())),
                                    preferred_element_type=jnp.float32)

    @pl.when(k == NB - 1)
    def _finish():
        d = d_run[...]
        inv = jnp.where(d > 0.0, 1.0 / d, 0.0)
        pooled = acc[...] * inv
        out = jax.lax.dot_general(pooled, wf_ref[...],
                                  (((1,), (0,)), ((), ())),
                                  preferred_element_type=jnp.float32)
        out_ref[...] = out + jnp.where(d > 0.0, bf_ref[...], 0.0)


@jax.jit
def kernel(feat, Wg, bg, Wf, bf, segment_ids):
    del bg
    seg3 = segment_ids.astype(jnp.int32).reshape(NB, 1, BN)
    bf2 = bf.reshape(1, D_FEAT)
    Wg = Wg.astype(jnp.bfloat16)
    out = pl.pallas_call(
        _pool_kernel,
        grid=(NB,),
        in_specs=[
            pl.BlockSpec((BN, D_FEAT), lambda k: (k, 0)),
            pl.BlockSpec((1, 1, BN), lambda k: (k, 0, 0)),
            pl.BlockSpec((D_FEAT, 1), lambda k: (0, 0)),
            pl.BlockSpec((D_FEAT, D_FEAT), lambda k: (0, 0)),
            pl.BlockSpec((1, D_FEAT), lambda k: (0, 0)),
        ],
        out_specs=pl.BlockSpec((NUM_SEGMENTS, D_FEAT), lambda k: (0, 0)),
        out_shape=jax.ShapeDtypeStruct((NUM_SEGMENTS, D_FEAT), jnp.float32),
        scratch_shapes=[
            pltpu.SMEM((1, 1), jnp.float32),
            pltpu.VMEM((NUM_SEGMENTS, 1), jnp.float32),
            pltpu.VMEM((NUM_SEGMENTS, D_FEAT), jnp.float32),
        ],
    )(feat, seg3, Wg, Wf, bf2)
    return out


# speedup vs baseline: 1.6414x; 1.0743x over previous
import jax
import jax.numpy as jnp
from jax.experimental import pallas as pl
from jax.experimental.pallas import tpu as pltpu

N_NODES = 50000
D_FEAT = 512
NUM_SEGMENTS = 256
BN = 5000
NB = N_NODES // BN
NEG = -1e30


def _pool_kernel(feat_ref, seg_ref, wg_ref, wf_ref, bf_ref, out_ref,
                 m_run, d_run, acc):
    k = pl.program_id(0)

    @pl.when(k == 0)
    def _init():
        m_run[0, 0] = NEG
        d_run[...] = jnp.zeros((NUM_SEGMENTS, 1), jnp.float32)
        acc[...] = jnp.zeros((NUM_SEGMENTS, D_FEAT), jnp.float32)

    feat = feat_ref[...].astype(jnp.bfloat16)
    seg = seg_ref[0]
    g_col = jax.lax.dot_general(feat, wg_ref[...],
                                (((1,), (0,)), ((), ())),
                                preferred_element_type=jnp.float32)
    g = jax.lax.transpose(g_col, (1, 0))
    m_old = m_run[0, 0]
    m_new = jnp.maximum(m_old, jnp.max(g))

    @pl.when(m_new > m_old)
    def _rescale():
        s_old = jnp.exp(m_old - m_new)
        d_run[...] = d_run[...] * s_old
        acc[...] = acc[...] * s_old
        m_run[0, 0] = m_new

    e = jnp.exp(g - m_new)
    rows = jax.lax.broadcasted_iota(jnp.int32, (NUM_SEGMENTS, BN), 0)
    w32 = jnp.where(seg == rows, e, 0.0)
    w = w32.astype(jnp.bfloat16)
    d_run[...] += jnp.sum(w32, axis=1, keepdims=True)
    acc[...] += jax.lax.dot_general(w, feat, (((1,), (0,)), ((), ())),
                                    preferred_element_type=jnp.float32)

    @pl.when(k == NB - 1)
    def _finish():
        d = d_run[...]
        inv = jnp.where(d > 0.0, 1.0 / d, 0.0)
        pooled = acc[...] * inv
        out = jax.lax.dot_general(pooled, wf_ref[...],
                                  (((1,), (0,)), ((), ())),
                                  preferred_element_type=jnp.float32)
        out_ref[...] = out + jnp.where(d > 0.0, bf_ref[...], 0.0)


@jax.jit
def kernel(feat, Wg, bg, Wf, bf, segment_ids):
    del bg
    seg3 = segment_ids.astype(jnp.int32).reshape(NB, 1, BN)
    bf2 = bf.reshape(1, D_FEAT)
    Wg = Wg.astype(jnp.bfloat16)
    out = pl.pallas_call(
        _pool_kernel,
        grid=(NB,),
        in_specs=[
            pl.BlockSpec((BN, D_FEAT), lambda k: (k, 0)),
            pl.BlockSpec((1, 1, BN), lambda k: (k, 0, 0)),
            pl.BlockSpec((D_FEAT, 1), lambda k: (0, 0)),
            pl.BlockSpec((D_FEAT, D_FEAT), lambda k: (0, 0)),
            pl.BlockSpec((1, D_FEAT), lambda k: (0, 0)),
        ],
        out_specs=pl.BlockSpec((NUM_SEGMENTS, D_FEAT), lambda k: (0, 0)),
        out_shape=jax.ShapeDtypeStruct((NUM_SEGMENTS, D_FEAT), jnp.float32),
        scratch_shapes=[
            pltpu.SMEM((1, 1), jnp.float32),
            pltpu.VMEM((NUM_SEGMENTS, 1), jnp.float32),
            pltpu.VMEM((NUM_SEGMENTS, D_FEAT), jnp.float32),
        ],
    )(feat, seg3, Wg, Wf, bf2)
    return out


# speedup vs baseline: 1.6568x; 1.0094x over previous
import jax
import jax.numpy as jnp
from jax.experimental import pallas as pl
from jax.experimental.pallas import tpu as pltpu

N_NODES = 50000
D_FEAT = 512
NUM_SEGMENTS = 256
BN = 10000
NB = N_NODES // BN
NEG = -1e30


def _pool_kernel(feat_ref, seg_ref, wg_ref, wf_ref, bf_ref, out_ref,
                 m_run, d_run, acc):
    k = pl.program_id(0)

    @pl.when(k == 0)
    def _init():
        m_run[0, 0] = NEG
        d_run[...] = jnp.zeros((NUM_SEGMENTS, 1), jnp.float32)
        acc[...] = jnp.zeros((NUM_SEGMENTS, D_FEAT), jnp.float32)

    feat = feat_ref[...].astype(jnp.bfloat16)
    seg = seg_ref[0]
    g_col = jax.lax.dot_general(feat, wg_ref[...],
                                (((1,), (0,)), ((), ())),
                                preferred_element_type=jnp.float32)
    g = jax.lax.transpose(g_col, (1, 0))
    m_old = m_run[0, 0]
    m_new = jnp.maximum(m_old, jnp.max(g))

    @pl.when(m_new > m_old)
    def _rescale():
        s_old = jnp.exp(m_old - m_new)
        d_run[...] = d_run[...] * s_old
        acc[...] = acc[...] * s_old
        m_run[0, 0] = m_new

    e = jnp.exp(g - m_new)
    rows = jax.lax.broadcasted_iota(jnp.int32, (NUM_SEGMENTS, BN), 0)
    w32 = jnp.where(seg == rows, e, 0.0)
    w = w32.astype(jnp.bfloat16)
    d_run[...] += jnp.sum(w32, axis=1, keepdims=True)
    acc[...] += jax.lax.dot_general(w, feat, (((1,), (0,)), ((), ())),
                                    preferred_element_type=jnp.float32)

    @pl.when(k == NB - 1)
    def _finish():
        d = d_run[...]
        inv = jnp.where(d > 0.0, 1.0 / d, 0.0)
        pooled = acc[...] * inv
        out = jax.lax.dot_general(pooled, wf_ref[...],
                                  (((1,), (0,)), ((), ())),
                                  preferred_element_type=jnp.float32)
        out_ref[...] = out + jnp.where(d > 0.0, bf_ref[...], 0.0)


@jax.jit
def kernel(feat, Wg, bg, Wf, bf, segment_ids):
    del bg
    seg3 = segment_ids.astype(jnp.int32).reshape(NB, 1, BN)
    bf2 = bf.reshape(1, D_FEAT)
    Wg = Wg.astype(jnp.bfloat16)
    out = pl.pallas_call(
        _pool_kernel,
        grid=(NB,),
        in_specs=[
            pl.BlockSpec((BN, D_FEAT), lambda k: (k, 0)),
            pl.BlockSpec((1, 1, BN), lambda k: (k, 0, 0)),
            pl.BlockSpec((D_FEAT, 1), lambda k: (0, 0)),
            pl.BlockSpec((D_FEAT, D_FEAT), lambda k: (0, 0)),
            pl.BlockSpec((1, D_FEAT), lambda k: (0, 0)),
        ],
        out_specs=pl.BlockSpec((NUM_SEGMENTS, D_FEAT), lambda k: (0, 0)),
        out_shape=jax.ShapeDtypeStruct((NUM_SEGMENTS, D_FEAT), jnp.float32),
        scratch_shapes=[
            pltpu.SMEM((1, 1), jnp.float32),
            pltpu.VMEM((NUM_SEGMENTS, 1), jnp.float32),
            pltpu.VMEM((NUM_SEGMENTS, D_FEAT), jnp.float32),
        ],
    )(feat, seg3, Wg, Wf, bf2)
    return out
